# baseline (speedup 1.0000x reference)
import jax
import jax.numpy as jnp
from jax.experimental import pallas as pl
from jax.experimental.pallas import tpu as pltpu

D = 128
N = 32
BATCH = 4
S = 256
E = 256
NCLS = 10
CIN = N * D
F32 = jnp.float32
BF16 = jnp.bfloat16


def _relu(x):
    return jnp.maximum(x, 0.0)


def _dot(a, b):
    return jnp.dot(a, b, preferred_element_type=F32)


def _shift_down(p):
    return jnp.concatenate([jnp.zeros_like(p[:1]), p[:-1]], axis=0)


def _shift_up(p):
    return jnp.concatenate([p[1:], jnp.zeros_like(p[:1])], axis=0)


def _conv_s1(x, w0, w1, w2):
    return _shift_down(_dot(x, w0)) + _dot(x, w1) + _shift_up(_dot(x, w2))


def _conv_s2(x, w01, w2):
    L, C = x.shape
    v = x.reshape(L // 2, 2 * C)
    p01 = _dot(v, w01)
    p2 = _dot(v[:, :C], w2)
    return p01 + _shift_up(p2)


def _up2(x):
    L, C = x.shape
    return jnp.broadcast_to(x[:, None, :], (L, 2, C)).reshape(2 * L, C)


def _body(xt_ref, edg_ref, w0_ref, b0_ref, we_ref, be_ref,
          k1_ref, k1b_ref, k2_ref, kd1_ref, kd2_ref, ko_ref, out_ref):
    src = edg_ref[0:1, :]
    dst = edg_ref[1:2, :]
    ni = jax.lax.broadcasted_iota(jnp.int32, (N, E), 0)
    dst_oh = (ni == dst).astype(F32)
    src_oh = (ni == src).astype(F32)
    A = jax.lax.dot_general(dst_oh, src_oh, (((1,), (1,)), ((), ())),
                            preferred_element_type=F32)
    r = jax.lax.broadcasted_iota(jnp.int32, (N, N), 0)
    c = jax.lax.broadcasted_iota(jnp.int32, (N, N), 1)
    M = (A + (r == c).astype(F32)).astype(BF16)

    KJ = 4
    ik = jax.lax.broadcasted_iota(jnp.int32, (KJ, KJ), 0)
    jk = jax.lax.broadcasted_iota(jnp.int32, (KJ, KJ), 1)
    eyek = (ik == jk).astype(BF16)
    Mk = (M[:, None, :, None] * eyek[None, :, None, :]).reshape(N * KJ, N * KJ)

    def _wmul(h3, w):
        return jax.lax.dot_general(h3, w, (((2,), (0,)), ((), ())),
                                   preferred_element_type=F32)

    def _mmul(mk, z3):
        zk = z3.reshape(N * KJ, S // KJ, D)
        t = jax.lax.dot_general(mk, zk, (((1,), (0,)), ((), ())),
                                preferred_element_type=F32)
        return t.reshape(N, S, D)

    x0 = xt_ref[0]
    z = jax.lax.dot_general(x0, w0_ref[...], (((1,), (0,)), ((), ())),
                            preferred_element_type=F32)
    h = _relu(_mmul(Mk, z.astype(BF16)) + b0_ref[...].reshape(1, 1, D))
    for i in range(4):
        z = _wmul(h.astype(BF16), we_ref[i])
        h = _relu(_mmul(Mk, z.astype(BF16)) + be_ref[i:i + 1, :].reshape(1, 1, D))

    hc = jnp.swapaxes(h, 0, 1).reshape(S, CIN)

    e1 = _relu(_conv_s2(hc, k1_ref[...], k1b_ref[...]))
    e2 = _relu(_conv_s2(e1, k2_ref[0], k2_ref[1, :256]))
    u1 = _up2(e2)
    d1 = _relu(_conv_s1(u1, kd1_ref[0, :256], kd1_ref[1, :256], kd1_ref[2, :256])
               + _conv_s1(e1, kd1_ref[0, 256:], kd1_ref[1, 256:], kd1_ref[2, 256:]))
    u2 = _up2(d1)
    d2 = _relu(_conv_s1(u2, kd2_ref[0, :256], kd2_ref[1, :256], kd2_ref[2, :256])
               + _conv_s1(hc, kd2_ref[0, 256:], kd2_ref[1, 256:], kd2_ref[2, 256:]))
    out_ref[0] = _dot(d2, ko_ref[...])


def kernel(x_, edges, W0, b0, W_enc, b_enc, K1, K2, Kd1, Kd2, Kout):
    xt = jnp.transpose(x_, (0, 2, 3, 1))
    b0r = b0.reshape(1, D)
    web = W_enc.astype(BF16)
    k1t = jnp.transpose(K1, (2, 1, 0))
    k1m = jnp.concatenate([k1t[0], k1t[1]], axis=0)
    k2t = jnp.transpose(K2, (2, 1, 0))
    k2r = jnp.stack([jnp.concatenate([k2t[0], k2t[1]], axis=0),
                     jnp.pad(k2t[2], ((0, 256), (0, 0)))])
    kd1t = jnp.transpose(Kd1, (2, 1, 0))
    kd2t = jnp.transpose(Kd2, (2, 1, 0))
    kot = Kout[:, :, 0].T

    whole = lambda shape: pl.BlockSpec(shape, lambda b: (0,) * len(shape))
    out = pl.pallas_call(
        _body,
        grid=(BATCH,),
        in_specs=[
            pl.BlockSpec((1, N, 2, S), lambda b: (b, 0, 0, 0)),
            whole((2, E)),
            whole((2, D)),
            whole((1, D)),
            whole((4, D, D)),
            whole((4, D)),
            whole((CIN * 2, 256)),
            whole((CIN, 256)),
            whole((2, 512, 256)),
            whole((3, 512, 256)),
            whole((3, 256 + CIN, 256)),
            whole((256, NCLS)),
        ],
        out_specs=pl.BlockSpec((1, S, NCLS), lambda b: (b, 0, 0)),
        out_shape=jax.ShapeDtypeStruct((BATCH, S, NCLS), F32),
        compiler_params=pltpu.CompilerParams(
            vmem_limit_bytes=100 * 1024 * 1024,
        ),
    )(xt, edges, W0, b0r, web, b_enc, k1m, k1t[2], k2r, kd1t, kd2t, kot)
    return jnp.transpose(out, (0, 2, 1))


# speedup vs baseline: 1.2173x; 1.2173x over previous
import jax
import jax.numpy as jnp
from jax.experimental import pallas as pl
from jax.experimental.pallas import tpu as pltpu

D = 128
N = 32
BATCH = 4
S = 256
E = 256
NCLS = 10
CIN = N * D
F32 = jnp.float32
BF16 = jnp.bfloat16


def _relu(x):
    return jnp.maximum(x, 0.0)


def _dot(a, b):
    return jnp.dot(a, b, preferred_element_type=F32)


def _shift_down(p):
    return jnp.concatenate([jnp.zeros_like(p[:1]), p[:-1]], axis=0)


def _shift_up(p):
    return jnp.concatenate([p[1:], jnp.zeros_like(p[:1])], axis=0)


def _conv_s1(x, w0, w1, w2):
    return _shift_down(_dot(x, w0)) + _dot(x, w1) + _shift_up(_dot(x, w2))


def _conv_s2(x, w01, w2):
    L, C = x.shape
    v = x.reshape(L // 2, 2 * C)
    p01 = _dot(v, w01)
    p2 = _dot(v[:, :C], w2)
    return p01 + _shift_up(p2)


def _up2(x):
    L, C = x.shape
    return jnp.broadcast_to(x[:, None, :], (L, 2, C)).reshape(2 * L, C)


def _body(xt_ref, edg_ref, w0_ref, b0_ref, we_ref, be_ref,
          k1_ref, k1b_ref, k2_ref, kd1_ref, kd2_ref, ko_ref, out_ref):
    src = edg_ref[0:1, :]
    dst = edg_ref[1:2, :]
    ni = jax.lax.broadcasted_iota(jnp.int32, (N, E), 0)
    dst_oh = (ni == dst).astype(F32)
    src_oh = (ni == src).astype(F32)
    A = jax.lax.dot_general(dst_oh, src_oh, (((1,), (1,)), ((), ())),
                            preferred_element_type=F32)
    r = jax.lax.broadcasted_iota(jnp.int32, (N, N), 0)
    c = jax.lax.broadcasted_iota(jnp.int32, (N, N), 1)
    M = (A + (r == c).astype(F32)).astype(BF16)

    def _wmul(h3, w):
        return jax.lax.dot_general(h3, w, (((2,), (0,)), ((), ())),
                                   preferred_element_type=F32)

    def _mmul(m, z3):
        return jax.lax.dot_general(m, z3, (((1,), (0,)), ((), ())),
                                   preferred_element_type=F32)

    x0 = xt_ref[0]
    z = jax.lax.dot_general(x0, w0_ref[...], (((1,), (0,)), ((), ())),
                            preferred_element_type=F32)
    h = _relu(_mmul(M, z.astype(BF16)) + b0_ref[...].reshape(1, 1, D))
    for i in range(4):
        z = _wmul(h.astype(BF16), we_ref[i])
        h = _relu(_mmul(M, z.astype(BF16)) + be_ref[i:i + 1, :].reshape(1, 1, D))

    hc = jnp.swapaxes(h, 0, 1).reshape(S, CIN).astype(BF16)

    e1 = _relu(_conv_s2(hc, k1_ref[...], k1b_ref[...])).astype(BF16)
    e2 = _relu(_conv_s2(e1, k2_ref[0], k2_ref[1, :256])).astype(BF16)
    u1 = _up2(e2)
    d1 = _relu(_conv_s1(u1, kd1_ref[0, :256], kd1_ref[1, :256], kd1_ref[2, :256])
               + _conv_s1(e1, kd1_ref[0, 256:], kd1_ref[1, 256:], kd1_ref[2, 256:])).astype(BF16)
    u2 = _up2(d1)
    d2 = _relu(_conv_s1(u2, kd2_ref[0, :256], kd2_ref[1, :256], kd2_ref[2, :256])
               + _conv_s1(hc, kd2_ref[0, 256:], kd2_ref[1, 256:], kd2_ref[2, 256:])).astype(BF16)
    out_ref[0] = _dot(d2, ko_ref[...])


def kernel(x_, edges, W0, b0, W_enc, b_enc, K1, K2, Kd1, Kd2, Kout):
    xt = jnp.transpose(x_, (0, 2, 3, 1))
    b0r = b0.reshape(1, D)
    web = W_enc.astype(BF16)
    k1t = jnp.transpose(K1.astype(BF16), (2, 1, 0))
    k1m = jnp.concatenate([k1t[0], k1t[1]], axis=0)
    k2t = jnp.transpose(K2.astype(BF16), (2, 1, 0))
    k2r = jnp.stack([jnp.concatenate([k2t[0], k2t[1]], axis=0),
                     jnp.pad(k2t[2], ((0, 256), (0, 0)))])
    kd1t = jnp.transpose(Kd1.astype(BF16), (2, 1, 0))
    kd2t = jnp.transpose(Kd2.astype(BF16), (2, 1, 0))
    kot = Kout[:, :, 0].T.astype(BF16)

    whole = lambda shape: pl.BlockSpec(shape, lambda b: (0,) * len(shape))
    out = pl.pallas_call(
        _body,
        grid=(BATCH,),
        in_specs=[
            pl.BlockSpec((1, N, 2, S), lambda b: (b, 0, 0, 0)),
            whole((2, E)),
            whole((2, D)),
            whole((1, D)),
            whole((4, D, D)),
            whole((4, D)),
            whole((CIN * 2, 256)),
            whole((CIN, 256)),
            whole((2, 512, 256)),
            whole((3, 512, 256)),
            whole((3, 256 + CIN, 256)),
            whole((256, NCLS)),
        ],
        out_specs=pl.BlockSpec((1, S, NCLS), lambda b: (b, 0, 0)),
        out_shape=jax.ShapeDtypeStruct((BATCH, S, NCLS), F32),
        compiler_params=pltpu.CompilerParams(
            vmem_limit_bytes=100 * 1024 * 1024,
        ),
    )(xt, edges, W0, b0r, web, b_enc, k1m, k1t[2], k2r, kd1t, kd2t, kot)
    return jnp.transpose(out, (0, 2, 1))


# speedup vs baseline: 1.2692x; 1.0426x over previous
import jax
import jax.numpy as jnp
from jax.experimental import pallas as pl
from jax.experimental.pallas import tpu as pltpu

D = 128
N = 32
BATCH = 4
S = 256
E = 256
NCLS = 10
CIN = N * D
F32 = jnp.float32
BF16 = jnp.bfloat16


def _relu(x):
    return jnp.maximum(x, 0.0)


def _dot(a, b):
    return jnp.dot(a, b, preferred_element_type=F32)


def _shift_down(p):
    return jnp.concatenate([jnp.zeros_like(p[:1]), p[:-1]], axis=0)


def _shift_up(p):
    return jnp.concatenate([p[1:], jnp.zeros_like(p[:1])], axis=0)


def _conv_s1(x, w0, w1, w2):
    return _shift_down(_dot(x, w0)) + _dot(x, w1) + _shift_up(_dot(x, w2))


def _conv_s2(x, w01, w2):
    L, C = x.shape
    v = x.reshape(L // 2, 2 * C)
    p01 = _dot(v, w01)
    p2 = _dot(v[:, :C], w2)
    return p01 + _shift_up(p2)


def _up2(x):
    L, C = x.shape
    return jnp.broadcast_to(x[:, None, :], (L, 2, C)).reshape(2 * L, C)


def _body(xt_ref, edg_ref, w0_ref, b0_ref, we_ref, be_ref,
          k1_ref, k1b_ref, k2_ref, kd1_ref, kd2_ref, ko_ref, out_ref):
    src = edg_ref[0:1, :]
    dst = edg_ref[1:2, :]
    ni = jax.lax.broadcasted_iota(jnp.int32, (N, E), 0)
    dst_oh = (ni == dst).astype(F32)
    src_oh = (ni == src).astype(F32)
    A = jax.lax.dot_general(dst_oh, src_oh, (((1,), (1,)), ((), ())),
                            preferred_element_type=F32)
    r = jax.lax.broadcasted_iota(jnp.int32, (N, N), 0)
    c = jax.lax.broadcasted_iota(jnp.int32, (N, N), 1)
    M = (A + (r == c).astype(F32)).astype(BF16)

    def _wmul(h3, w):
        z = jax.lax.dot_general(h3.reshape(N * S, D), w, (((1,), (0,)), ((), ())),
                                preferred_element_type=F32)
        return z.reshape(N, S, D).astype(BF16)

    def _mmul(m, z3):
        return jax.lax.dot_general(m, z3, (((1,), (0,)), ((), ())),
                                   preferred_element_type=F32)

    x0 = xt_ref[0]
    z = jax.lax.dot_general(x0, w0_ref[...], (((1,), (0,)), ((), ())),
                            preferred_element_type=F32).astype(BF16)
    h = _relu(_mmul(M, z) + b0_ref[...].reshape(1, 1, D)).astype(BF16)
    for i in range(4):
        h = _relu(_mmul(M, _wmul(h, we_ref[i]))
                  + be_ref[i:i + 1, :].reshape(1, 1, D)).astype(BF16)

    hc = jnp.swapaxes(h, 0, 1).reshape(S, CIN)

    e1 = _relu(_conv_s2(hc, k1_ref[...], k1b_ref[...])).astype(BF16)
    e2 = _relu(_conv_s2(e1, k2_ref[0], k2_ref[1, :256])).astype(BF16)
    u1 = _up2(e2)
    d1 = _relu(_conv_s1(u1, kd1_ref[0, :256], kd1_ref[1, :256], kd1_ref[2, :256])
               + _conv_s1(e1, kd1_ref[0, 256:], kd1_ref[1, 256:], kd1_ref[2, 256:])).astype(BF16)
    u2 = _up2(d1)
    d2 = _relu(_conv_s1(u2, kd2_ref[0, :256], kd2_ref[1, :256], kd2_ref[2, :256])
               + _conv_s1(hc, kd2_ref[0, 256:], kd2_ref[1, 256:], kd2_ref[2, 256:])).astype(BF16)
    out_ref[0] = _dot(d2, ko_ref[...])


def kernel(x_, edges, W0, b0, W_enc, b_enc, K1, K2, Kd1, Kd2, Kout):
    xt = jnp.transpose(x_, (0, 2, 3, 1))
    b0r = b0.reshape(1, D)
    web = W_enc.astype(BF16)
    k1t = jnp.transpose(K1.astype(BF16), (2, 1, 0))
    k1m = jnp.concatenate([k1t[0], k1t[1]], axis=0)
    k2t = jnp.transpose(K2.astype(BF16), (2, 1, 0))
    k2r = jnp.stack([jnp.concatenate([k2t[0], k2t[1]], axis=0),
                     jnp.pad(k2t[2], ((0, 256), (0, 0)))])
    kd1t = jnp.transpose(Kd1.astype(BF16), (2, 1, 0))
    kd2t = jnp.transpose(Kd2.astype(BF16), (2, 1, 0))
    kot = Kout[:, :, 0].T.astype(BF16)

    whole = lambda shape: pl.BlockSpec(shape, lambda b: (0,) * len(shape))
    out = pl.pallas_call(
        _body,
        grid=(BATCH,),
        in_specs=[
            pl.BlockSpec((1, N, 2, S), lambda b: (b, 0, 0, 0)),
            whole((2, E)),
            whole((2, D)),
            whole((1, D)),
            whole((4, D, D)),
            whole((4, D)),
            whole((CIN * 2, 256)),
            whole((CIN, 256)),
            whole((2, 512, 256)),
            whole((3, 512, 256)),
            whole((3, 256 + CIN, 256)),
            whole((256, NCLS)),
        ],
        out_specs=pl.BlockSpec((1, S, NCLS), lambda b: (b, 0, 0)),
        out_shape=jax.ShapeDtypeStruct((BATCH, S, NCLS), F32),
        compiler_params=pltpu.CompilerParams(
            vmem_limit_bytes=100 * 1024 * 1024,
        ),
    )(xt, edges, W0, b0r, web, b_enc, k1m, k1t[2], k2r, kd1t, kd2t, kot)
    return jnp.transpose(out, (0, 2, 1))


# speedup vs baseline: 1.2927x; 1.0185x over previous
import jax
import jax.numpy as jnp
from jax.experimental import pallas as pl
from jax.experimental.pallas import tpu as pltpu

D = 128
N = 32
BATCH = 4
S = 256
E = 256
NCLS = 10
CIN = N * D
F32 = jnp.float32
BF16 = jnp.bfloat16


def _relu(x):
    return jnp.maximum(x, 0.0)


def _dot(a, b):
    return jnp.dot(a, b, preferred_element_type=F32)


def _shift_down(p):
    return jnp.concatenate([jnp.zeros_like(p[:1]), p[:-1]], axis=0)


def _shift_up(p):
    return jnp.concatenate([p[1:], jnp.zeros_like(p[:1])], axis=0)


def _conv_s1(x, w0, w1, w2):
    return _shift_down(_dot(x, w0)) + _dot(x, w1) + _shift_up(_dot(x, w2))


def _conv_s2(x, w01, w2):
    L, C = x.shape
    v = x.reshape(L // 2, 2 * C)
    p01 = _dot(v, w01)
    p2 = _dot(v[:, :C], w2)
    return p01 + _shift_up(p2)


def _up2(x):
    L, C = x.shape
    return jnp.broadcast_to(x[:, None, :], (L, 2, C)).reshape(2 * L, C)


def _body(xt_ref, edg_ref, w0_ref, b0_ref, we_ref, be_ref,
          k1_hbm, k1b_hbm, k2_ref, kd1_ref, kd2_hbm, ko_ref, out_ref,
          k1_ref, k1b_ref, kd2_ref, sem1, sem2, sem3):
    cp1 = pltpu.make_async_copy(k1_hbm, k1_ref, sem1)
    cp2 = pltpu.make_async_copy(k1b_hbm, k1b_ref, sem2)
    cp3 = pltpu.make_async_copy(kd2_hbm, kd2_ref, sem3)
    first = pl.program_id(0) == 0

    @pl.when(first)
    def _start_weight_copies():
        cp1.start()
        cp2.start()
        cp3.start()

    src = edg_ref[0:1, :]
    dst = edg_ref[1:2, :]
    ni = jax.lax.broadcasted_iota(jnp.int32, (N, E), 0)
    dst_oh = (ni == dst).astype(F32)
    src_oh = (ni == src).astype(F32)
    A = jax.lax.dot_general(dst_oh, src_oh, (((1,), (1,)), ((), ())),
                            preferred_element_type=F32)
    r = jax.lax.broadcasted_iota(jnp.int32, (N, N), 0)
    c = jax.lax.broadcasted_iota(jnp.int32, (N, N), 1)
    M = (A + (r == c).astype(F32)).astype(BF16)

    def _wmul(h3, w):
        z = jax.lax.dot_general(h3.reshape(N * S, D), w, (((1,), (0,)), ((), ())),
                                preferred_element_type=F32)
        return z.reshape(N, S, D).astype(BF16)

    def _mmul(m, z3):
        return jax.lax.dot_general(m, z3, (((1,), (0,)), ((), ())),
                                   preferred_element_type=F32)

    x0 = xt_ref[0]
    z = jax.lax.dot_general(x0, w0_ref[...], (((1,), (0,)), ((), ())),
                            preferred_element_type=F32).astype(BF16)
    h = _relu(_mmul(M, z) + b0_ref[...].reshape(1, 1, D)).astype(BF16)
    for i in range(4):
        h = _relu(_mmul(M, _wmul(h, we_ref[i]))
                  + be_ref[i:i + 1, :].reshape(1, 1, D)).astype(BF16)

    hc = jnp.swapaxes(h, 0, 1).reshape(S, CIN)

    @pl.when(first)
    def _wait_k1():
        cp1.wait()
        cp2.wait()

    e1 = _relu(_conv_s2(hc, k1_ref[...], k1b_ref[...])).astype(BF16)
    e2 = _relu(_conv_s2(e1, k2_ref[0], k2_ref[1, :256])).astype(BF16)
    u1 = _up2(e2)
    d1 = _relu(_conv_s1(u1, kd1_ref[0, :256], kd1_ref[1, :256], kd1_ref[2, :256])
               + _conv_s1(e1, kd1_ref[0, 256:], kd1_ref[1, 256:], kd1_ref[2, 256:])).astype(BF16)
    u2 = _up2(d1)

    @pl.when(first)
    def _wait_kd2():
        cp3.wait()

    d2 = _relu(_conv_s1(u2, kd2_ref[0, :256], kd2_ref[1, :256], kd2_ref[2, :256])
               + _conv_s1(hc, kd2_ref[0, 256:], kd2_ref[1, 256:], kd2_ref[2, 256:])).astype(BF16)
    out_ref[0] = _dot(d2, ko_ref[...])


def kernel(x_, edges, W0, b0, W_enc, b_enc, K1, K2, Kd1, Kd2, Kout):
    xt = jnp.transpose(x_, (0, 2, 3, 1))
    b0r = b0.reshape(1, D)
    web = W_enc.astype(BF16)
    k1t = jnp.transpose(K1.astype(BF16), (2, 1, 0))
    k1m = jnp.concatenate([k1t[0], k1t[1]], axis=0)
    k2t = jnp.transpose(K2.astype(BF16), (2, 1, 0))
    k2r = jnp.stack([jnp.concatenate([k2t[0], k2t[1]], axis=0),
                     jnp.pad(k2t[2], ((0, 256), (0, 0)))])
    kd1t = jnp.transpose(Kd1.astype(BF16), (2, 1, 0))
    kd2t = jnp.transpose(Kd2.astype(BF16), (2, 1, 0))
    kot = Kout[:, :, 0].T.astype(BF16)

    whole = lambda shape: pl.BlockSpec(shape, lambda b: (0,) * len(shape))
    out = pl.pallas_call(
        _body,
        grid=(BATCH,),
        in_specs=[
            pl.BlockSpec((1, N, 2, S), lambda b: (b, 0, 0, 0)),
            whole((2, E)),
            whole((2, D)),
            whole((1, D)),
            whole((4, D, D)),
            whole((4, D)),
            pl.BlockSpec(memory_space=pltpu.MemorySpace.HBM),
            pl.BlockSpec(memory_space=pltpu.MemorySpace.HBM),
            whole((2, 512, 256)),
            whole((3, 512, 256)),
            pl.BlockSpec(memory_space=pltpu.MemorySpace.HBM),
            whole((256, NCLS)),
        ],
        out_specs=pl.BlockSpec((1, S, NCLS), lambda b: (b, 0, 0)),
        out_shape=jax.ShapeDtypeStruct((BATCH, S, NCLS), F32),
        scratch_shapes=[
            pltpu.VMEM((CIN * 2, 256), BF16),
            pltpu.VMEM((CIN, 256), BF16),
            pltpu.VMEM((3, 256 + CIN, 256), BF16),
            pltpu.SemaphoreType.DMA,
            pltpu.SemaphoreType.DMA,
            pltpu.SemaphoreType.DMA,
        ],
        compiler_params=pltpu.CompilerParams(
            vmem_limit_bytes=100 * 1024 * 1024,
        ),
    )(xt, edges, W0, b0r, web, b_enc, k1m, k1t[2], k2r, kd1t, kd2t, kot)
    return jnp.transpose(out, (0, 2, 1))
